# Initial kernel scaffold; baseline (speedup 1.0000x reference)
import jax
import jax.numpy as jnp
from jax.experimental import pallas as pl


def kernel(node_features, edge_indices, W1, b1, W2, b2, Wl, bl):
    raise NotImplementedError("write your pallas kernel here")


# speedup vs baseline: 18.9344x; 18.9344x over previous
import functools

import jax
import jax.numpy as jnp
from jax import lax
from jax.experimental import pallas as pl
from jax.experimental.pallas import tpu as pltpu
from jax.experimental.pallas import tpu_sc as plsc

_N = 10000
_E = 320000
_F = 128
_NC = 2
_NS = 16
_NW = _NC * _NS
_EPW = _E // _NW
_CB = 80
_NCH = _EPW // _CB
_NP = 10240
_RPT = _NP // _NS

_mesh = plsc.VectorSubcoreMesh(core_axis_name="c", subcore_axis_name="s")


@functools.partial(
    pl.kernel,
    out_type=jax.ShapeDtypeStruct((_NC, _NP), jnp.float32),
    mesh=_mesh,
    scratch_types=[
        pltpu.VMEM((_NCH, _CB), jnp.int32),
        pltpu.VMEM((_CB,), jnp.float32),
        pltpu.VMEM_SHARED((_NP,), jnp.float32),
    ],
)
def _deg_kernel(dst_hbm, zeros_hbm, out_hbm, dst_v, ones_v, acc):
    c = lax.axis_index("c")
    s = lax.axis_index("s")
    w = s * _NC + c
    pltpu.sync_copy(dst_hbm.at[w], dst_v)
    for i in range(_CB // 16):
        ones_v[pl.ds(16 * i, 16)] = jnp.full((16,), 1.0, jnp.float32)
    pltpu.sync_copy(zeros_hbm.at[pl.ds(s * _RPT, _RPT)],
                    acc.at[pl.ds(s * _RPT, _RPT)])
    plsc.subcore_barrier()

    def body(j, carry):
        pltpu.sync_copy(ones_v, acc.at[dst_v.at[j]], add=True)
        return carry

    lax.fori_loop(0, _NCH, body, 0)
    plsc.subcore_barrier()
    pltpu.sync_copy(acc.at[pl.ds(s * _RPT, _RPT)],
                    out_hbm.at[c, pl.ds(s * _RPT, _RPT)])


_IBC = 25
_NIB = _NCH // _IBC


@functools.partial(
    pl.kernel,
    out_type=jax.ShapeDtypeStruct((_NC, _NP, _F), jnp.float32),
    mesh=_mesh,
    scratch_types=[
        pltpu.VMEM((_IBC, _CB), jnp.int32),
        pltpu.VMEM((_IBC, _CB), jnp.int32),
        pltpu.VMEM((2, _CB, _F), jnp.float32),
        pltpu.SemaphoreType.DMA,
        pltpu.VMEM_SHARED((_NP, _F), jnp.float32),
    ],
)
def _prop_kernel(xs_hbm, src_hbm, dst_hbm, zeros_hbm, out_hbm,
                 src_v, dst_v, buf, sem, acc):
    c = lax.axis_index("c")
    s = lax.axis_index("s")
    w = s * _NC + c
    pltpu.sync_copy(zeros_hbm.at[pl.ds(s * _RPT, _RPT)],
                    acc.at[pl.ds(s * _RPT, _RPT)])
    plsc.subcore_barrier()

    def blk_body(blk, carry):
        pltpu.sync_copy(src_hbm.at[w * _NIB + blk], src_v)
        pltpu.sync_copy(dst_hbm.at[w * _NIB + blk], dst_v)

        def body(j, inner):
            pltpu.async_copy(xs_hbm.at[src_v.at[j]], buf.at[0], sem).wait()
            pltpu.sync_copy(buf.at[0], acc.at[dst_v.at[j]], add=True)
            return inner

        lax.fori_loop(0, _IBC, body, 0)
        return carry

    lax.fori_loop(0, _NIB, blk_body, 0)
    plsc.subcore_barrier()
    pltpu.sync_copy(acc.at[pl.ds(s * _RPT, _RPT)],
                    out_hbm.at[c, pl.ds(s * _RPT, _RPT)])


def _tc_first(x_ref, w_ref, deg_ref, xs_ref, dinv_ref):
    dinv = lax.rsqrt(deg_ref[...])
    h = jnp.dot(x_ref[...], w_ref[...], preferred_element_type=jnp.float32)
    dinv_ref[...] = dinv
    xs_ref[...] = dinv * h


def _tc_mid(za_ref, zb_ref, xs_ref, dinv_ref, b_ref, w_ref, out_ref):
    z = za_ref[...] + zb_ref[...] + xs_ref[...]
    x1 = jnp.maximum(dinv_ref[...] * z + b_ref[...], 0.0)
    h = jnp.dot(x1, w_ref[...], preferred_element_type=jnp.float32)
    out_ref[...] = dinv_ref[...] * h


def _tc_last(za_ref, zb_ref, xs_ref, dinv_ref, b_ref, wl_ref, bl_ref, out_ref):
    z = za_ref[...] + zb_ref[...] + xs_ref[...]
    x2 = jnp.maximum(dinv_ref[...] * z + b_ref[...], 0.0)
    out_ref[...] = (jnp.dot(x2, wl_ref[...], preferred_element_type=jnp.float32)
                    + bl_ref[...])


def kernel(node_features, edge_indices, W1, b1, W2, b2, Wl, bl):
    ei = edge_indices.astype(jnp.int32)
    src = ei[0].reshape(_NW * _NIB, _IBC, _CB)
    dst = ei[1].reshape(_NW, _NCH, _CB)
    dst_blk = ei[1].reshape(_NW * _NIB, _IBC, _CB)
    zeros_f = jnp.zeros((_NP, _F), jnp.float32)
    zeros_1 = jnp.zeros((_NP,), jnp.float32)

    degp = _deg_kernel(dst, zeros_1)
    deg_col = (degp[0, :_N] + degp[1, :_N] + 1.0)[:, None]

    xs1, dinv = pl.pallas_call(
        _tc_first,
        out_shape=[
            jax.ShapeDtypeStruct((_N, _F), jnp.float32),
            jax.ShapeDtypeStruct((_N, 1), jnp.float32),
        ],
    )(node_features, W1, deg_col)

    z1 = _prop_kernel(xs1, src, dst_blk, zeros_f)

    xs2 = pl.pallas_call(
        _tc_mid,
        out_shape=jax.ShapeDtypeStruct((_N, _F), jnp.float32),
    )(z1[0, :_N], z1[1, :_N], xs1, dinv, b1.reshape(1, _F), W2)

    z2 = _prop_kernel(xs2, src, dst_blk, zeros_f)

    out = pl.pallas_call(
        _tc_last,
        out_shape=jax.ShapeDtypeStruct((_N, 40), jnp.float32),
    )(z2[0, :_N], z2[1, :_N], xs2, dinv, b2.reshape(1, _F), Wl,
      bl.reshape(1, 40))
    return out


# speedup vs baseline: 27.6338x; 1.4594x over previous
import functools

import jax
import jax.numpy as jnp
from jax import lax
from jax.experimental import pallas as pl
from jax.experimental.pallas import tpu as pltpu
from jax.experimental.pallas import tpu_sc as plsc

_N = 10000
_E = 320000
_F = 128
_NC = 2
_NS = 16
_NW = _NC * _NS
_EPW = _E // _NW
_CB = 80
_NCH = _EPW // _CB
_NP = 10240
_RPT = _NP // _NS

_mesh = plsc.VectorSubcoreMesh(core_axis_name="c", subcore_axis_name="s")


@functools.partial(
    pl.kernel,
    out_type=jax.ShapeDtypeStruct((_NC, _NP), jnp.float32),
    mesh=_mesh,
    scratch_types=[
        pltpu.VMEM((_NCH, _CB), jnp.int32),
        pltpu.VMEM((_CB,), jnp.float32),
        pltpu.VMEM_SHARED((_NP,), jnp.float32),
    ],
)
def _deg_kernel(dst_hbm, zeros_hbm, out_hbm, dst_v, ones_v, acc):
    c = lax.axis_index("c")
    s = lax.axis_index("s")
    w = s * _NC + c
    pltpu.sync_copy(dst_hbm.at[w], dst_v)
    for i in range(_CB // 16):
        ones_v[pl.ds(16 * i, 16)] = jnp.full((16,), 1.0, jnp.float32)
    pltpu.sync_copy(zeros_hbm.at[pl.ds(s * _RPT, _RPT)],
                    acc.at[pl.ds(s * _RPT, _RPT)])
    plsc.subcore_barrier()

    def body(j, carry):
        pltpu.sync_copy(ones_v, acc.at[dst_v.at[j]], add=True)
        return carry

    lax.fori_loop(0, _NCH, body, 0)
    plsc.subcore_barrier()
    pltpu.sync_copy(acc.at[pl.ds(s * _RPT, _RPT)],
                    out_hbm.at[c, pl.ds(s * _RPT, _RPT)])


_IBC = 25
_NIB = _NCH // _IBC


@functools.partial(
    pl.kernel,
    out_type=jax.ShapeDtypeStruct((_NC, _NP, _F), jnp.float32),
    mesh=_mesh,
    scratch_types=[
        pltpu.VMEM((_IBC, _CB), jnp.int32),
        pltpu.VMEM((_IBC, _CB), jnp.int32),
        pltpu.VMEM((2, _CB, _F), jnp.float32),
        pltpu.SemaphoreType.DMA,
        pltpu.SemaphoreType.DMA,
        pltpu.VMEM_SHARED((_NP, _F), jnp.float32),
    ],
)
def _prop_kernel(xs_hbm, src_hbm, dst_hbm, zeros_hbm, out_hbm,
                 src_v, dst_v, buf, sem0, sem1, acc):
    c = lax.axis_index("c")
    s = lax.axis_index("s")
    w = s * _NC + c
    pltpu.sync_copy(zeros_hbm.at[pl.ds(s * _RPT, _RPT)],
                    acc.at[pl.ds(s * _RPT, _RPT)])
    plsc.subcore_barrier()

    sems = (sem0, sem1)

    def gissue(jj, b):
        pltpu.async_copy(xs_hbm.at[src_v.at[jj]], buf.at[b], sems[b])

    def gwait(jj, b):
        pltpu.make_async_copy(xs_hbm.at[src_v.at[jj]], buf.at[b],
                              sems[b]).wait()

    def scat(jj, b):
        pltpu.sync_copy(buf.at[b], acc.at[dst_v.at[jj]], add=True)

    def blk_body(blk, carry):
        pltpu.sync_copy(src_hbm.at[w * _NIB + blk], src_v)
        pltpu.sync_copy(dst_hbm.at[w * _NIB + blk], dst_v)
        gissue(0, 0)

        def pair(j2, inner):
            j = 2 * j2
            gissue(j + 1, 1)
            gwait(j, 0)
            scat(j, 0)
            gissue(j + 2, 0)
            gwait(j + 1, 1)
            scat(j + 1, 1)
            return inner

        lax.fori_loop(0, (_IBC - 1) // 2, pair, 0)
        gwait(_IBC - 1, 0)
        scat(_IBC - 1, 0)
        return carry

    lax.fori_loop(0, _NIB, blk_body, 0)
    plsc.subcore_barrier()
    pltpu.sync_copy(acc.at[pl.ds(s * _RPT, _RPT)],
                    out_hbm.at[c, pl.ds(s * _RPT, _RPT)])


def _tc_first(x_ref, w_ref, deg_ref, xs_ref, dinv_ref):
    dinv = lax.rsqrt(deg_ref[...])
    h = jnp.dot(x_ref[...], w_ref[...], preferred_element_type=jnp.float32)
    dinv_ref[...] = dinv
    xs_ref[...] = dinv * h


def _tc_mid(za_ref, zb_ref, xs_ref, dinv_ref, b_ref, w_ref, out_ref):
    z = za_ref[...] + zb_ref[...] + xs_ref[...]
    x1 = jnp.maximum(dinv_ref[...] * z + b_ref[...], 0.0)
    h = jnp.dot(x1, w_ref[...], preferred_element_type=jnp.float32)
    out_ref[...] = dinv_ref[...] * h


def _tc_last(za_ref, zb_ref, xs_ref, dinv_ref, b_ref, wl_ref, bl_ref, out_ref):
    z = za_ref[...] + zb_ref[...] + xs_ref[...]
    x2 = jnp.maximum(dinv_ref[...] * z + b_ref[...], 0.0)
    out_ref[...] = (jnp.dot(x2, wl_ref[...], preferred_element_type=jnp.float32)
                    + bl_ref[...])


def kernel(node_features, edge_indices, W1, b1, W2, b2, Wl, bl):
    ei = edge_indices.astype(jnp.int32)
    src = ei[0].reshape(_NW * _NIB, _IBC, _CB)
    dst = ei[1].reshape(_NW, _NCH, _CB)
    dst_blk = ei[1].reshape(_NW * _NIB, _IBC, _CB)
    zeros_f = jnp.zeros((_NP, _F), jnp.float32)
    zeros_1 = jnp.zeros((_NP,), jnp.float32)

    degp = _deg_kernel(dst, zeros_1)
    deg_col = (degp[0, :_N] + degp[1, :_N] + 1.0)[:, None]

    xs1, dinv = pl.pallas_call(
        _tc_first,
        out_shape=[
            jax.ShapeDtypeStruct((_N, _F), jnp.float32),
            jax.ShapeDtypeStruct((_N, 1), jnp.float32),
        ],
    )(node_features, W1, deg_col)

    z1 = _prop_kernel(xs1, src, dst_blk, zeros_f)

    xs2 = pl.pallas_call(
        _tc_mid,
        out_shape=jax.ShapeDtypeStruct((_N, _F), jnp.float32),
    )(z1[0, :_N], z1[1, :_N], xs1, dinv, b1.reshape(1, _F), W2)

    z2 = _prop_kernel(xs2, src, dst_blk, zeros_f)

    out = pl.pallas_call(
        _tc_last,
        out_shape=jax.ShapeDtypeStruct((_N, 40), jnp.float32),
    )(z2[0, :_N], z2[1, :_N], xs2, dinv, b2.reshape(1, _F), Wl,
      bl.reshape(1, 40))
    return out
